# baseline (speedup 1.0000x reference)
import functools

import jax
import jax.numpy as jnp
from jax import lax
from jax.experimental import pallas as pl
from jax.experimental.pallas import tpu as pltpu
from jax.experimental.pallas import tpu_sc as plsc

N = 10000
D = 128
E = 320000
NP = 10240
EP = 327680
NC, NS = 2, 16
R = NP // NS
RCH = 128
NRCH = R // RCH
ET = EP // NS
W = 128
NCHK = ET // W
DW = 1024
ALPHA = 0.1
NITER = 4

_f32 = jnp.float32
_i32 = jnp.int32


def _frsqrt(x):
    i = lax.bitcast_convert_type(x, _i32)
    i = jnp.int32(0x5F3759DF) - jnp.right_shift(i, 1)
    y = lax.bitcast_convert_type(i, _f32)
    xh = 0.5 * x
    for _ in range(3):
        y = y * (1.5 - xh * y * y)
    return y


NBUF = 5


def _body(res_hbm, srce_hbm, dste_hbm, out_hbm, g_hbm,
          acc_sp, degs_sp, degd_sp,
          sidx, didx, rows, ones, z1,
          dtmp, nsb, ndb, mb, *sems):
    gsem = sems[:NBUF]
    ssem = sems[NBUF:]
    c = lax.axis_index("c")
    s = lax.axis_index("s")
    node_base = c * NP
    row0 = s * R
    zero16 = jnp.zeros((16,), _f32)
    one16 = jnp.ones((16,), _f32)

    zbuf = rows.at[4]
    accb = (rows.at[0], rows.at[1])
    mixb = (rows.at[2], rows.at[3])
    mixbuf = mixb[0]

    def zfill():
        @pl.loop(0, RCH)
        def _(i):
            for k in range(4):
                zbuf[i, pl.ds(16 * k, 16)] = zero16

    zfill()
    for k in range(DW // 16):
        ones[pl.ds(16 * k, 16)] = one16

    @pl.loop(0, R // 16)
    def _(i):
        z1[pl.ds(16 * i, 16)] = zero16

    @pl.loop(0, NRCH)
    def _(j):
        pltpu.sync_copy(zbuf, acc_sp.at[pl.ds(row0 + j * RCH, RCH)])

    pltpu.sync_copy(z1, degs_sp.at[pl.ds(row0, R)])
    pltpu.sync_copy(z1, degd_sp.at[pl.ds(row0, R)])

    pltpu.sync_copy(srce_hbm.at[s], sidx)
    pltpu.sync_copy(dste_hbm.at[s], didx)

    plsc.subcore_barrier()

    with jax.named_scope("phase_deg"):
        @pl.loop(0, ET // DW)
        def _(j):
            ds_ = degs_sp.at[sidx.at[pl.ds(j * DW, DW)]]
            dd_ = degd_sp.at[didx.at[pl.ds(j * DW, DW)]]
            pltpu.async_copy(ones, ds_, gsem[0], add=True)
            pltpu.async_copy(ones, dd_, gsem[1], add=True)
            pltpu.make_async_copy(ones, ds_, gsem[0]).wait()
            pltpu.make_async_copy(ones, dd_, gsem[1]).wait()

    @pl.loop(0, ET // 16)
    def _(i):
        sl = pl.ds(16 * i, 16)
        sidx[sl] = sidx[sl] + node_base

    plsc.subcore_barrier()

    pltpu.sync_copy(degs_sp.at[pl.ds(row0, R)], dtmp)

    @pl.loop(0, R // 16)
    def _(i):
        sl = pl.ds(16 * i, 16)
        nsb[sl] = _frsqrt(jnp.maximum(dtmp[sl], 1.0))

    pltpu.sync_copy(degd_sp.at[pl.ds(row0, R)], dtmp)

    @pl.loop(0, R // 16)
    def _(i):
        sl = pl.ds(16 * i, 16)
        nd = _frsqrt(jnp.maximum(dtmp[sl], 1.0))
        ndb[sl] = nd
        mb[sl] = (1.0 - ALPHA) * nsb[sl] * nd

    @pl.loop(0, NRCH)
    def _(j):
        st = row0 + j * RCH
        pltpu.sync_copy(res_hbm.at[pl.ds(node_base + st, RCH)], mixbuf)

        @pl.loop(0, RCH)
        def _(i):
            ri = jnp.full((16,), j * RCH + i, _i32)
            sc = plsc.load_gather(nsb, [ri])
            for k in range(4):
                sl = pl.ds(16 * k, 16)
                mixbuf[i, sl] = mixbuf[i, sl] * sc

        pltpu.sync_copy(mixbuf, g_hbm.at[pl.ds(node_base + st, RCH)])

    plsc.subcore_barrier()

    for it in range(NITER):
        last = it == NITER - 1

        with jax.named_scope("phase_scatter"):
            def sslice(ref, j):
                return ref.at[pl.ds(j * W, W)]

            for b in range(NBUF):
                pltpu.async_copy(
                    g_hbm.at[sslice(sidx, b)], rows.at[b], gsem[b])

            @pl.loop(0, NCHK, step=NBUF)
            def _(base):
                for b in range(NBUF):
                    j = base + b
                    bp = (b - 1) % NBUF
                    pltpu.make_async_copy(
                        g_hbm.at[sslice(sidx, j)], rows.at[b], gsem[b]).wait()
                    pltpu.async_copy(
                        rows.at[b], acc_sp.at[sslice(didx, j)], ssem[b],
                        add=True)

                    @pl.when(j >= 1)
                    def _():
                        pltpu.make_async_copy(
                            rows.at[bp], acc_sp.at[sslice(didx, j - 1)],
                            ssem[bp]).wait()
                        nj = j + NBUF - 1

                        @pl.when(nj < NCHK)
                        def _():
                            pltpu.async_copy(
                                g_hbm.at[sslice(sidx, nj)], rows.at[bp],
                                gsem[bp])

            pltpu.make_async_copy(
                rows.at[(NCHK - 1) % NBUF],
                acc_sp.at[sslice(didx, NCHK - 1)],
                ssem[(NCHK - 1) % NBUF]).wait()

        plsc.subcore_barrier()

        with jax.named_scope("phase_mix"):
            if not last:
                zfill()

            def acc_src(j):
                return acc_sp.at[pl.ds(row0 + j * RCH, RCH)]

            def res_src(j):
                return res_hbm.at[pl.ds(node_base + row0 + j * RCH, RCH)]

            def mix_dst(j):
                dref = out_hbm if last else g_hbm
                return dref.at[pl.ds(node_base + row0 + j * RCH, RCH)]

            ars = (ssem[0], ssem[1])
            rrs = (ssem[2], ssem[3])
            wrs = (gsem[0], gsem[1])
            pltpu.async_copy(acc_src(0), accb[0], ars[0])
            pltpu.async_copy(res_src(0), mixb[0], rrs[0])
            for j in range(NRCH):
                p = j & 1
                mp, ap = mixb[p], accb[p]
                pltpu.make_async_copy(acc_src(j), ap, ars[p]).wait()
                if not last:
                    pltpu.sync_copy(zbuf, acc_src(j))
                pltpu.make_async_copy(res_src(j), mp, rrs[p]).wait()
                if j + 1 < NRCH:
                    q = 1 - p
                    if j >= 1:
                        pltpu.make_async_copy(
                            mixb[q], mix_dst(j - 1), wrs[q]).wait()
                    pltpu.async_copy(acc_src(j + 1), accb[q], ars[q])
                    pltpu.async_copy(res_src(j + 1), mixb[q], rrs[q])

                @pl.loop(0, RCH)
                def _(i):
                    ri = jnp.full((16,), j * RCH + i, _i32)
                    if last:
                        a = (1.0 - ALPHA) * plsc.load_gather(ndb, [ri])
                        b = jnp.full((16,), ALPHA, _f32)
                    else:
                        a = plsc.load_gather(mb, [ri])
                        b = ALPHA * plsc.load_gather(nsb, [ri])
                    for k in range(4):
                        sl = pl.ds(16 * k, 16)
                        mp[i, sl] = b * mp[i, sl] + a * ap[i, sl]

                pltpu.async_copy(mp, mix_dst(j), wrs[p])

            for j in (NRCH - 2, NRCH - 1):
                pltpu.make_async_copy(mixb[j & 1], mix_dst(j), wrs[j & 1]).wait()

        if not last:
            plsc.subcore_barrier()


_mesh = plsc.VectorSubcoreMesh(
    core_axis_name="c", subcore_axis_name="s", num_cores=NC, num_subcores=NS)

_sc_call = functools.partial(
    pl.kernel,
    out_type=(
        jax.ShapeDtypeStruct((NC * NP, 64), _f32),
        jax.ShapeDtypeStruct((NC * NP, 64), _f32),
    ),
    mesh=_mesh,
    compiler_params=pltpu.CompilerParams(needs_layout_passes=False, use_tc_tiling_on_sc=False),
    scratch_types=[
        pltpu.VMEM_SHARED((NP, 64), _f32),
        pltpu.VMEM_SHARED((NP,), _f32),
        pltpu.VMEM_SHARED((NP,), _f32),
        pltpu.VMEM((ET,), _i32),
        pltpu.VMEM((ET,), _i32),
        pltpu.VMEM((NBUF, W, 64), _f32),
        pltpu.VMEM((DW,), _f32),
        pltpu.VMEM((R,), _f32),
        pltpu.VMEM((R,), _f32),
        pltpu.VMEM((R,), _f32),
        pltpu.VMEM((R,), _f32),
        pltpu.VMEM((R,), _f32),
    ] + [pltpu.SemaphoreType.DMA] * (2 * NBUF),
)(_body)


def kernel(res, edge_index):
    src = edge_index[0]
    dst = edge_index[1]
    npad = EP - E
    pad_idx = N + (jnp.arange(npad, dtype=_i32) % (NP - N))
    srcp = jnp.concatenate([src, pad_idx]).reshape(NS, ET)
    dstp = jnp.concatenate([dst, pad_idx]).reshape(NS, ET)
    rz = jnp.zeros((NP - N, 64), _f32)
    res_pack = jnp.concatenate([res[:, :64], rz, res[:, 64:], rz], axis=0)
    out_pack, _ = _sc_call(res_pack, srcp, dstp)
    return jnp.concatenate([out_pack[:N], out_pack[NP:NP + N]], axis=1)


# speedup vs baseline: 1.0239x; 1.0239x over previous
import functools

import jax
import jax.numpy as jnp
from jax import lax
from jax.experimental import pallas as pl
from jax.experimental.pallas import tpu as pltpu
from jax.experimental.pallas import tpu_sc as plsc

N = 10000
D = 128
E = 320000
NP = 10240
EP = 327680
NC, NS = 2, 16
R = NP // NS
RCH = 128
NRCH = R // RCH
ET = EP // NS
W = 128
NCHK = ET // W
DW = 1024
ALPHA = 0.1
NITER = 4

_f32 = jnp.float32
_i32 = jnp.int32


def _frsqrt(x):
    i = lax.bitcast_convert_type(x, _i32)
    i = jnp.int32(0x5F3759DF) - jnp.right_shift(i, 1)
    y = lax.bitcast_convert_type(i, _f32)
    xh = 0.5 * x
    for _ in range(3):
        y = y * (1.5 - xh * y * y)
    return y


NBUF = 5


def _body(res_hbm, srce_hbm, dste_hbm, out_hbm, g_hbm,
          acc_sp, degs_sp, degd_sp,
          sidx, didx, rows, ones, z1,
          dtmp, nsb, ndb, mb, *sems):
    gsem = sems[:NBUF]
    ssem = sems[NBUF:]
    c = lax.axis_index("c")
    s = lax.axis_index("s")
    node_base = c * NP
    row0 = s * R
    zero16 = jnp.zeros((16,), _f32)
    one16 = jnp.ones((16,), _f32)

    zbuf = rows.at[4]
    accb = (rows.at[0], rows.at[1])
    mixb = (rows.at[2], rows.at[3])
    mixbuf = mixb[0]

    def zfill():
        @pl.loop(0, RCH)
        def _(i):
            for k in range(4):
                zbuf[i, pl.ds(16 * k, 16)] = zero16

    zfill()
    for k in range(DW // 16):
        ones[pl.ds(16 * k, 16)] = one16

    @pl.loop(0, R // 16)
    def _(i):
        z1[pl.ds(16 * i, 16)] = zero16

    @pl.loop(0, NRCH)
    def _(j):
        pltpu.sync_copy(zbuf, acc_sp.at[pl.ds(row0 + j * RCH, RCH)])

    pltpu.sync_copy(z1, degs_sp.at[pl.ds(row0, R)])
    pltpu.sync_copy(z1, degd_sp.at[pl.ds(row0, R)])

    pltpu.sync_copy(srce_hbm.at[s], sidx)
    pltpu.sync_copy(dste_hbm.at[s], didx)

    plsc.subcore_barrier()

    with jax.named_scope("phase_deg"):
        @pl.loop(0, ET // DW)
        def _(j):
            ds_ = degs_sp.at[sidx.at[pl.ds(j * DW, DW)]]
            dd_ = degd_sp.at[didx.at[pl.ds(j * DW, DW)]]
            pltpu.async_copy(ones, ds_, gsem[0], add=True)
            pltpu.async_copy(ones, dd_, gsem[1], add=True)
            pltpu.make_async_copy(ones, ds_, gsem[0]).wait()
            pltpu.make_async_copy(ones, dd_, gsem[1]).wait()

    @pl.loop(0, ET // 16)
    def _(i):
        sl = pl.ds(16 * i, 16)
        sidx[sl] = sidx[sl] + node_base

    plsc.subcore_barrier()

    pltpu.sync_copy(degs_sp.at[pl.ds(row0, R)], dtmp)

    @pl.loop(0, R // 16)
    def _(i):
        sl = pl.ds(16 * i, 16)
        nsb[sl] = _frsqrt(jnp.maximum(dtmp[sl], 1.0))

    pltpu.sync_copy(degd_sp.at[pl.ds(row0, R)], dtmp)

    @pl.loop(0, R // 16)
    def _(i):
        sl = pl.ds(16 * i, 16)
        nd = _frsqrt(jnp.maximum(dtmp[sl], 1.0))
        ndb[sl] = nd
        mb[sl] = (1.0 - ALPHA) * nsb[sl] * nd

    def g0_src(j):
        return res_hbm.at[pl.ds(node_base + row0 + j * RCH, RCH)]

    def g0_dst(j):
        return g_hbm.at[pl.ds(node_base + row0 + j * RCH, RCH)]

    pltpu.async_copy(g0_src(0), mixb[0], ssem[2])
    for j in range(NRCH):
        p = j & 1
        mp = mixb[p]
        pltpu.make_async_copy(g0_src(j), mp, ssem[2 + p]).wait()
        if j + 1 < NRCH:
            q = 1 - p
            if j >= 1:
                pltpu.make_async_copy(mixb[q], g0_dst(j - 1), gsem[q]).wait()
            pltpu.async_copy(g0_src(j + 1), mixb[q], ssem[2 + q])

        @pl.loop(0, RCH)
        def _(i):
            ri = jnp.full((16,), j * RCH + i, _i32)
            sc = plsc.load_gather(nsb, [ri])
            for k in range(4):
                sl = pl.ds(16 * k, 16)
                mp[i, sl] = mp[i, sl] * sc

        pltpu.async_copy(mp, g0_dst(j), gsem[p])

    for j in (NRCH - 2, NRCH - 1):
        pltpu.make_async_copy(mixb[j & 1], g0_dst(j), gsem[j & 1]).wait()

    plsc.subcore_barrier()

    for it in range(NITER):
        last = it == NITER - 1

        with jax.named_scope("phase_scatter"):
            def sslice(ref, j):
                return ref.at[pl.ds(j * W, W)]

            for b in range(NBUF):
                pltpu.async_copy(
                    g_hbm.at[sslice(sidx, b)], rows.at[b], gsem[b])

            @pl.loop(0, NCHK, step=NBUF)
            def _(base):
                for b in range(NBUF):
                    j = base + b
                    pltpu.make_async_copy(
                        g_hbm.at[sslice(sidx, j)], rows.at[b], gsem[b]).wait()
                    pltpu.sync_copy(
                        rows.at[b], acc_sp.at[sslice(didx, j)], add=True)
                    nj = j + NBUF

                    @pl.when(nj < NCHK)
                    def _():
                        pltpu.async_copy(
                            g_hbm.at[sslice(sidx, nj)], rows.at[b], gsem[b])

        plsc.subcore_barrier()

        with jax.named_scope("phase_mix"):
            if not last:
                zfill()

            def acc_src(j):
                return acc_sp.at[pl.ds(row0 + j * RCH, RCH)]

            def res_src(j):
                return res_hbm.at[pl.ds(node_base + row0 + j * RCH, RCH)]

            def mix_dst(j):
                dref = out_hbm if last else g_hbm
                return dref.at[pl.ds(node_base + row0 + j * RCH, RCH)]

            ars = (ssem[0], ssem[1])
            rrs = (ssem[2], ssem[3])
            wrs = (gsem[0], gsem[1])
            pltpu.async_copy(acc_src(0), accb[0], ars[0])
            pltpu.async_copy(res_src(0), mixb[0], rrs[0])
            for j in range(NRCH):
                p = j & 1
                mp, ap = mixb[p], accb[p]
                pltpu.make_async_copy(acc_src(j), ap, ars[p]).wait()
                if not last:
                    pltpu.sync_copy(zbuf, acc_src(j))
                pltpu.make_async_copy(res_src(j), mp, rrs[p]).wait()
                if j + 1 < NRCH:
                    q = 1 - p
                    if j >= 1:
                        pltpu.make_async_copy(
                            mixb[q], mix_dst(j - 1), wrs[q]).wait()
                    pltpu.async_copy(acc_src(j + 1), accb[q], ars[q])
                    pltpu.async_copy(res_src(j + 1), mixb[q], rrs[q])

                @pl.loop(0, RCH)
                def _(i):
                    ri = jnp.full((16,), j * RCH + i, _i32)
                    if last:
                        a = (1.0 - ALPHA) * plsc.load_gather(ndb, [ri])
                        b = jnp.full((16,), ALPHA, _f32)
                    else:
                        a = plsc.load_gather(mb, [ri])
                        b = ALPHA * plsc.load_gather(nsb, [ri])
                    for k in range(4):
                        sl = pl.ds(16 * k, 16)
                        mp[i, sl] = b * mp[i, sl] + a * ap[i, sl]

                pltpu.async_copy(mp, mix_dst(j), wrs[p])

            for j in (NRCH - 2, NRCH - 1):
                pltpu.make_async_copy(mixb[j & 1], mix_dst(j), wrs[j & 1]).wait()

        if not last:
            plsc.subcore_barrier()


_mesh = plsc.VectorSubcoreMesh(
    core_axis_name="c", subcore_axis_name="s", num_cores=NC, num_subcores=NS)

_sc_call = functools.partial(
    pl.kernel,
    out_type=(
        jax.ShapeDtypeStruct((NC * NP, 64), _f32),
        jax.ShapeDtypeStruct((NC * NP, 64), _f32),
    ),
    mesh=_mesh,
    compiler_params=pltpu.CompilerParams(needs_layout_passes=False, use_tc_tiling_on_sc=False),
    scratch_types=[
        pltpu.VMEM_SHARED((NP, 64), _f32),
        pltpu.VMEM_SHARED((NP,), _f32),
        pltpu.VMEM_SHARED((NP,), _f32),
        pltpu.VMEM((ET,), _i32),
        pltpu.VMEM((ET,), _i32),
        pltpu.VMEM((NBUF, W, 64), _f32),
        pltpu.VMEM((DW,), _f32),
        pltpu.VMEM((R,), _f32),
        pltpu.VMEM((R,), _f32),
        pltpu.VMEM((R,), _f32),
        pltpu.VMEM((R,), _f32),
        pltpu.VMEM((R,), _f32),
    ] + [pltpu.SemaphoreType.DMA] * (2 * NBUF),
)(_body)


def kernel(res, edge_index):
    src = edge_index[0]
    dst = edge_index[1]
    npad = EP - E
    pad_idx = N + (jnp.arange(npad, dtype=_i32) % (NP - N))
    srcp = jnp.concatenate([src, pad_idx]).reshape(NS, ET)
    dstp = jnp.concatenate([dst, pad_idx]).reshape(NS, ET)
    rz = jnp.zeros((NP - N, 64), _f32)
    res_pack = jnp.concatenate([res[:, :64], rz, res[:, 64:], rz], axis=0)
    out_pack, _ = _sc_call(res_pack, srcp, dstp)
    return jnp.concatenate([out_pack[:N], out_pack[NP:NP + N]], axis=1)


# speedup vs baseline: 1.0368x; 1.0125x over previous
import functools

import jax
import jax.numpy as jnp
from jax import lax
from jax.experimental import pallas as pl
from jax.experimental.pallas import tpu as pltpu
from jax.experimental.pallas import tpu_sc as plsc

N = 10000
D = 128
E = 320000
NP = 10240
EP = 327680
NC, NS = 2, 16
R = NP // NS
RCH = 128
NRCH = R // RCH
ET = EP // NS
W = 128
NCHK = ET // W
DW = 1024
ALPHA = 0.1
NITER = 4

_f32 = jnp.float32
_i32 = jnp.int32


def _frsqrt(x):
    i = lax.bitcast_convert_type(x, _i32)
    i = jnp.int32(0x5F3759DF) - jnp.right_shift(i, 1)
    y = lax.bitcast_convert_type(i, _f32)
    xh = 0.5 * x
    for _ in range(3):
        y = y * (1.5 - xh * y * y)
    return y


NBUF = 5


def _body(res_hbm, srce_hbm, dste_hbm, out_hbm, g_hbm,
          acc_sp, degs_sp, degd_sp,
          sidx, didx, rows, ones, z1,
          dtmp, nsb, ndb, mb, *sems):
    gsem = sems[:NBUF]
    ssem = sems[NBUF:]
    c = lax.axis_index("c")
    s = lax.axis_index("s")
    node_base = c * NP
    row0 = s * R
    zero16 = jnp.zeros((16,), _f32)
    one16 = jnp.ones((16,), _f32)

    zbuf = rows.at[4]
    accb = (rows.at[0], rows.at[1])
    mixb = (rows.at[2], rows.at[3])
    mixbuf = mixb[0]

    def zfill():
        @pl.loop(0, RCH)
        def _(i):
            for k in range(4):
                zbuf[i, pl.ds(16 * k, 16)] = zero16

    pltpu.async_copy(srce_hbm.at[s], sidx, ssem[0])
    pltpu.async_copy(dste_hbm.at[s], didx, ssem[1])

    zfill()
    for k in range(DW // 16):
        ones[pl.ds(16 * k, 16)] = one16

    @pl.loop(0, R // 16)
    def _(i):
        z1[pl.ds(16 * i, 16)] = zero16

    @pl.loop(0, NRCH)
    def _(j):
        pltpu.sync_copy(zbuf, acc_sp.at[pl.ds(row0 + j * RCH, RCH)])

    pltpu.sync_copy(z1, degs_sp.at[pl.ds(row0, R)])
    pltpu.sync_copy(z1, degd_sp.at[pl.ds(row0, R)])

    pltpu.make_async_copy(srce_hbm.at[s], sidx, ssem[0]).wait()
    pltpu.make_async_copy(dste_hbm.at[s], didx, ssem[1]).wait()

    plsc.subcore_barrier()

    with jax.named_scope("phase_deg"):
        @pl.loop(0, ET // DW)
        def _(j):
            ds_ = degs_sp.at[sidx.at[pl.ds(j * DW, DW)]]
            dd_ = degd_sp.at[didx.at[pl.ds(j * DW, DW)]]
            pltpu.async_copy(ones, ds_, gsem[0], add=True)
            pltpu.async_copy(ones, dd_, gsem[1], add=True)
            pltpu.make_async_copy(ones, ds_, gsem[0]).wait()
            pltpu.make_async_copy(ones, dd_, gsem[1]).wait()

    @pl.loop(0, ET // 16)
    def _(i):
        sl = pl.ds(16 * i, 16)
        sidx[sl] = sidx[sl] + node_base

    plsc.subcore_barrier()

    pltpu.sync_copy(degs_sp.at[pl.ds(row0, R)], dtmp)

    @pl.loop(0, R // 16)
    def _(i):
        sl = pl.ds(16 * i, 16)
        nsb[sl] = _frsqrt(jnp.maximum(dtmp[sl], 1.0))

    pltpu.sync_copy(degd_sp.at[pl.ds(row0, R)], dtmp)

    @pl.loop(0, R // 16)
    def _(i):
        sl = pl.ds(16 * i, 16)
        nd = _frsqrt(jnp.maximum(dtmp[sl], 1.0))
        ndb[sl] = nd
        mb[sl] = (1.0 - ALPHA) * nsb[sl] * nd

    def g0_src(j):
        return res_hbm.at[pl.ds(node_base + row0 + j * RCH, RCH)]

    def g0_dst(j):
        return g_hbm.at[pl.ds(node_base + row0 + j * RCH, RCH)]

    pltpu.async_copy(g0_src(0), mixb[0], ssem[2])
    for j in range(NRCH):
        p = j & 1
        mp = mixb[p]
        pltpu.make_async_copy(g0_src(j), mp, ssem[2 + p]).wait()
        if j + 1 < NRCH:
            q = 1 - p
            if j >= 1:
                pltpu.make_async_copy(mixb[q], g0_dst(j - 1), gsem[q]).wait()
            pltpu.async_copy(g0_src(j + 1), mixb[q], ssem[2 + q])

        @pl.loop(0, RCH)
        def _(i):
            ri = jnp.full((16,), j * RCH + i, _i32)
            sc = plsc.load_gather(nsb, [ri])
            for k in range(4):
                sl = pl.ds(16 * k, 16)
                mp[i, sl] = mp[i, sl] * sc

        pltpu.async_copy(mp, g0_dst(j), gsem[p])

    for j in (NRCH - 2, NRCH - 1):
        pltpu.make_async_copy(mixb[j & 1], g0_dst(j), gsem[j & 1]).wait()

    plsc.subcore_barrier()

    for it in range(NITER):
        last = it == NITER - 1

        with jax.named_scope("phase_scatter"):
            def sslice(ref, j):
                return ref.at[pl.ds(j * W, W)]

            for b in range(NBUF):
                pltpu.async_copy(
                    g_hbm.at[sslice(sidx, b)], rows.at[b], gsem[b])

            @pl.loop(0, NCHK, step=NBUF)
            def _(base):
                for b in range(NBUF):
                    j = base + b
                    pltpu.make_async_copy(
                        g_hbm.at[sslice(sidx, j)], rows.at[b], gsem[b]).wait()
                    pltpu.sync_copy(
                        rows.at[b], acc_sp.at[sslice(didx, j)], add=True)
                    nj = j + NBUF

                    @pl.when(nj < NCHK)
                    def _():
                        pltpu.async_copy(
                            g_hbm.at[sslice(sidx, nj)], rows.at[b], gsem[b])

        plsc.subcore_barrier()

        with jax.named_scope("phase_mix"):
            if not last:
                zfill()

            def acc_src(j):
                return acc_sp.at[pl.ds(row0 + j * RCH, RCH)]

            def res_src(j):
                return res_hbm.at[pl.ds(node_base + row0 + j * RCH, RCH)]

            def mix_dst(j):
                dref = out_hbm if last else g_hbm
                return dref.at[pl.ds(node_base + row0 + j * RCH, RCH)]

            ars = (ssem[0], ssem[1])
            rrs = (ssem[2], ssem[3])
            wrs = (gsem[0], gsem[1])
            pltpu.async_copy(acc_src(0), accb[0], ars[0])
            pltpu.async_copy(res_src(0), mixb[0], rrs[0])
            for j in range(NRCH):
                p = j & 1
                mp, ap = mixb[p], accb[p]
                pltpu.make_async_copy(acc_src(j), ap, ars[p]).wait()
                if not last:
                    pltpu.async_copy(zbuf, acc_src(j), gsem[2])
                pltpu.make_async_copy(res_src(j), mp, rrs[p]).wait()
                if j + 1 < NRCH:
                    q = 1 - p
                    if j >= 1:
                        pltpu.make_async_copy(
                            mixb[q], mix_dst(j - 1), wrs[q]).wait()
                    pltpu.async_copy(acc_src(j + 1), accb[q], ars[q])
                    pltpu.async_copy(res_src(j + 1), mixb[q], rrs[q])

                @pl.loop(0, RCH)
                def _(i):
                    ri = jnp.full((16,), j * RCH + i, _i32)
                    if last:
                        a = (1.0 - ALPHA) * plsc.load_gather(ndb, [ri])
                        b = jnp.full((16,), ALPHA, _f32)
                    else:
                        a = plsc.load_gather(mb, [ri])
                        b = ALPHA * plsc.load_gather(nsb, [ri])
                    for k in range(4):
                        sl = pl.ds(16 * k, 16)
                        mp[i, sl] = b * mp[i, sl] + a * ap[i, sl]

                pltpu.async_copy(mp, mix_dst(j), wrs[p])

            for j in (NRCH - 2, NRCH - 1):
                pltpu.make_async_copy(mixb[j & 1], mix_dst(j), wrs[j & 1]).wait()
            if not last:
                for j in range(NRCH):
                    pltpu.make_async_copy(zbuf, acc_src(j), gsem[2]).wait()

        if not last:
            plsc.subcore_barrier()


_mesh = plsc.VectorSubcoreMesh(
    core_axis_name="c", subcore_axis_name="s", num_cores=NC, num_subcores=NS)

_sc_call = functools.partial(
    pl.kernel,
    out_type=(
        jax.ShapeDtypeStruct((NC * NP, 64), _f32),
        jax.ShapeDtypeStruct((NC * NP, 64), _f32),
    ),
    mesh=_mesh,
    compiler_params=pltpu.CompilerParams(needs_layout_passes=False, use_tc_tiling_on_sc=False),
    scratch_types=[
        pltpu.VMEM_SHARED((NP, 64), _f32),
        pltpu.VMEM_SHARED((NP,), _f32),
        pltpu.VMEM_SHARED((NP,), _f32),
        pltpu.VMEM((ET,), _i32),
        pltpu.VMEM((ET,), _i32),
        pltpu.VMEM((NBUF, W, 64), _f32),
        pltpu.VMEM((DW,), _f32),
        pltpu.VMEM((R,), _f32),
        pltpu.VMEM((R,), _f32),
        pltpu.VMEM((R,), _f32),
        pltpu.VMEM((R,), _f32),
        pltpu.VMEM((R,), _f32),
    ] + [pltpu.SemaphoreType.DMA] * (2 * NBUF),
)(_body)


def kernel(res, edge_index):
    src = edge_index[0]
    dst = edge_index[1]
    npad = EP - E
    pad_idx = N + (jnp.arange(npad, dtype=_i32) % (NP - N))
    srcp = jnp.concatenate([src, pad_idx]).reshape(NS, ET)
    dstp = jnp.concatenate([dst, pad_idx]).reshape(NS, ET)
    rz = jnp.zeros((NP - N, 64), _f32)
    res_pack = jnp.concatenate([res[:, :64], rz, res[:, 64:], rz], axis=0)
    out_pack, _ = _sc_call(res_pack, srcp, dstp)
    return jnp.concatenate([out_pack[:N], out_pack[NP:NP + N]], axis=1)


# speedup vs baseline: 1.0421x; 1.0051x over previous
import functools

import jax
import jax.numpy as jnp
from jax import lax
from jax.experimental import pallas as pl
from jax.experimental.pallas import tpu as pltpu
from jax.experimental.pallas import tpu_sc as plsc

N = 10000
D = 128
E = 320000
NP = 10240
EP = 327680
NC, NS = 2, 16
R = NP // NS
RCH = 128
NRCH = R // RCH
ET = EP // NS
W = 128
NCHK = ET // W
DW = 1024
ALPHA = 0.1
NITER = 4

_f32 = jnp.float32
_i32 = jnp.int32


def _frsqrt(x):
    i = lax.bitcast_convert_type(x, _i32)
    i = jnp.int32(0x5F3759DF) - jnp.right_shift(i, 1)
    y = lax.bitcast_convert_type(i, _f32)
    xh = 0.5 * x
    for _ in range(3):
        y = y * (1.5 - xh * y * y)
    return y


NBUF = 5
RING = 4


def _body(res_hbm, srce_hbm, dste_hbm, out_hbm, g_hbm,
          acc_sp, degs_sp, degd_sp,
          sidx, didx, rows, ones, z1,
          dtmp, nsb, ndb, mb, *sems):
    gsem = sems[:NBUF]
    ssem = sems[NBUF:]
    c = lax.axis_index("c")
    s = lax.axis_index("s")
    node_base = c * NP
    row0 = s * R
    zero16 = jnp.zeros((16,), _f32)
    one16 = jnp.ones((16,), _f32)

    zbuf = rows.at[4]
    accb = (rows.at[0], rows.at[1])
    mixb = (rows.at[2], rows.at[3])
    mixbuf = mixb[0]

    def zfill():
        @pl.loop(0, RCH)
        def _(i):
            for k in range(4):
                zbuf[i, pl.ds(16 * k, 16)] = zero16

    pltpu.async_copy(srce_hbm.at[s], sidx, ssem[0])
    pltpu.async_copy(dste_hbm.at[s], didx, ssem[1])

    zfill()
    for k in range(DW // 16):
        ones[pl.ds(16 * k, 16)] = one16

    @pl.loop(0, R // 16)
    def _(i):
        z1[pl.ds(16 * i, 16)] = zero16

    @pl.loop(0, NRCH)
    def _(j):
        pltpu.sync_copy(zbuf, acc_sp.at[pl.ds(row0 + j * RCH, RCH)])

    pltpu.sync_copy(z1, degs_sp.at[pl.ds(row0, R)])
    pltpu.sync_copy(z1, degd_sp.at[pl.ds(row0, R)])

    pltpu.make_async_copy(srce_hbm.at[s], sidx, ssem[0]).wait()
    pltpu.make_async_copy(dste_hbm.at[s], didx, ssem[1]).wait()

    plsc.subcore_barrier()

    @pl.loop(0, ET // DW)
    def _(j):
        ds_ = degs_sp.at[sidx.at[pl.ds(j * DW, DW)]]
        dd_ = degd_sp.at[didx.at[pl.ds(j * DW, DW)]]
        pltpu.async_copy(ones, ds_, gsem[0], add=True)
        pltpu.async_copy(ones, dd_, gsem[1], add=True)
        pltpu.make_async_copy(ones, ds_, gsem[0]).wait()
        pltpu.make_async_copy(ones, dd_, gsem[1]).wait()

    @pl.loop(0, ET // 16)
    def _(i):
        sl = pl.ds(16 * i, 16)
        sidx[sl] = sidx[sl] + node_base

    plsc.subcore_barrier()

    pltpu.sync_copy(degs_sp.at[pl.ds(row0, R)], dtmp)

    @pl.loop(0, R // 16)
    def _(i):
        sl = pl.ds(16 * i, 16)
        nsb[sl] = _frsqrt(jnp.maximum(dtmp[sl], 1.0))

    pltpu.sync_copy(degd_sp.at[pl.ds(row0, R)], dtmp)

    @pl.loop(0, R // 16)
    def _(i):
        sl = pl.ds(16 * i, 16)
        nd = _frsqrt(jnp.maximum(dtmp[sl], 1.0))
        ndb[sl] = nd
        mb[sl] = (1.0 - ALPHA) * nsb[sl] * nd

    def g0_src(j):
        return res_hbm.at[pl.ds(node_base + row0 + j * RCH, RCH)]

    def g0_dst(j):
        return g_hbm.at[pl.ds(node_base + row0 + j * RCH, RCH)]

    pltpu.async_copy(g0_src(0), mixb[0], ssem[2])
    for j in range(NRCH):
        p = j & 1
        mp = mixb[p]
        pltpu.make_async_copy(g0_src(j), mp, ssem[2 + p]).wait()
        if j + 1 < NRCH:
            q = 1 - p
            if j >= 1:
                pltpu.make_async_copy(mixb[q], g0_dst(j - 1), gsem[q]).wait()
            pltpu.async_copy(g0_src(j + 1), mixb[q], ssem[2 + q])

        @pl.loop(0, RCH)
        def _(i):
            ri = jnp.full((16,), j * RCH + i, _i32)
            sc = plsc.load_gather(nsb, [ri])
            for k in range(4):
                sl = pl.ds(16 * k, 16)
                mp[i, sl] = mp[i, sl] * sc

        pltpu.async_copy(mp, g0_dst(j), gsem[p])

    for j in (NRCH - 2, NRCH - 1):
        pltpu.make_async_copy(mixb[j & 1], g0_dst(j), gsem[j & 1]).wait()

    plsc.subcore_barrier()

    for it in range(NITER):
        last = it == NITER - 1

        def sslice(ref, j):
            return ref.at[pl.ds(j * W, W)]

        for b in range(RING):
            pltpu.async_copy(
                g_hbm.at[sslice(sidx, b)], rows.at[b], gsem[b])

        @pl.loop(0, NCHK, step=RING)
        def _(base):
            for b in range(RING):
                j = base + b
                pltpu.make_async_copy(
                    g_hbm.at[sslice(sidx, j)], rows.at[b], gsem[b]).wait()
                pltpu.sync_copy(
                    rows.at[b], acc_sp.at[sslice(didx, j)], add=True)
                nj = j + RING

                @pl.when(nj < NCHK)
                def _():
                    pltpu.async_copy(
                        g_hbm.at[sslice(sidx, nj)], rows.at[b], gsem[b])

        plsc.subcore_barrier()

        def acc_src(j):
            return acc_sp.at[pl.ds(row0 + j * RCH, RCH)]

        def res_src(j):
            return res_hbm.at[pl.ds(node_base + row0 + j * RCH, RCH)]

        def mix_dst(j):
            dref = out_hbm if last else g_hbm
            return dref.at[pl.ds(node_base + row0 + j * RCH, RCH)]

        ars = (ssem[0], ssem[1])
        rrs = (ssem[2], ssem[3])
        wrs = (gsem[0], gsem[1])
        pltpu.async_copy(acc_src(0), accb[0], ars[0])
        pltpu.async_copy(res_src(0), mixb[0], rrs[0])
        for j in range(NRCH):
            p = j & 1
            mp, ap = mixb[p], accb[p]
            pltpu.make_async_copy(acc_src(j), ap, ars[p]).wait()
            if not last:
                pltpu.async_copy(zbuf, acc_src(j), gsem[2])
            pltpu.make_async_copy(res_src(j), mp, rrs[p]).wait()
            if j + 1 < NRCH:
                q = 1 - p
                if j >= 1:
                    pltpu.make_async_copy(
                        mixb[q], mix_dst(j - 1), wrs[q]).wait()
                pltpu.async_copy(acc_src(j + 1), accb[q], ars[q])
                pltpu.async_copy(res_src(j + 1), mixb[q], rrs[q])

            @pl.loop(0, RCH)
            def _(i):
                ri = jnp.full((16,), j * RCH + i, _i32)
                if last:
                    a = (1.0 - ALPHA) * plsc.load_gather(ndb, [ri])
                    b = jnp.full((16,), ALPHA, _f32)
                else:
                    a = plsc.load_gather(mb, [ri])
                    b = ALPHA * plsc.load_gather(nsb, [ri])
                for k in range(4):
                    sl = pl.ds(16 * k, 16)
                    mp[i, sl] = b * mp[i, sl] + a * ap[i, sl]

            pltpu.async_copy(mp, mix_dst(j), wrs[p])

        for j in (NRCH - 2, NRCH - 1):
            pltpu.make_async_copy(mixb[j & 1], mix_dst(j), wrs[j & 1]).wait()
        if not last:
            for j in range(NRCH):
                pltpu.make_async_copy(zbuf, acc_src(j), gsem[2]).wait()

        if not last:
            plsc.subcore_barrier()


_mesh = plsc.VectorSubcoreMesh(
    core_axis_name="c", subcore_axis_name="s", num_cores=NC, num_subcores=NS)

_sc_call = functools.partial(
    pl.kernel,
    out_type=(
        jax.ShapeDtypeStruct((NC * NP, 64), _f32),
        jax.ShapeDtypeStruct((NC * NP, 64), _f32),
    ),
    mesh=_mesh,
    compiler_params=pltpu.CompilerParams(needs_layout_passes=False, use_tc_tiling_on_sc=False),
    scratch_types=[
        pltpu.VMEM_SHARED((NP, 64), _f32),
        pltpu.VMEM_SHARED((NP,), _f32),
        pltpu.VMEM_SHARED((NP,), _f32),
        pltpu.VMEM((ET,), _i32),
        pltpu.VMEM((ET,), _i32),
        pltpu.VMEM((NBUF, W, 64), _f32),
        pltpu.VMEM((DW,), _f32),
        pltpu.VMEM((R,), _f32),
        pltpu.VMEM((R,), _f32),
        pltpu.VMEM((R,), _f32),
        pltpu.VMEM((R,), _f32),
        pltpu.VMEM((R,), _f32),
    ] + [pltpu.SemaphoreType.DMA] * (2 * NBUF),
)(_body)


def kernel(res, edge_index):
    src = edge_index[0]
    dst = edge_index[1]
    npad = EP - E
    pad_idx = N + (jnp.arange(npad, dtype=_i32) % (NP - N))
    srcp = jnp.concatenate([src, pad_idx]).reshape(NS, ET)
    dstp = jnp.concatenate([dst, pad_idx]).reshape(NS, ET)
    rz = jnp.zeros((NP - N, 64), _f32)
    res_pack = jnp.concatenate([res[:, :64], rz, res[:, 64:], rz], axis=0)
    out_pack, _ = _sc_call(res_pack, srcp, dstp)
    return jnp.concatenate([out_pack[:N], out_pack[NP:NP + N]], axis=1)
